# baseline (speedup 1.0000x reference)
import functools

import jax
import jax.numpy as jnp
from jax.experimental import pallas as pl
from jax.experimental.pallas import tpu as pltpu


def _cdiv(a, b):
    return (a + b - 1) // b


def _masked_k_dot(a_ref, t, nw, k0):
    a = a_ref[...].astype(t.dtype)
    a_head = a[:, :k0]
    a_tail = a[:, k0:]
    col = k0 + jax.lax.broadcasted_iota(jnp.int32, a_tail.shape, 1)
    a_tail = jnp.where(col < nw, a_tail, 0)
    acc = jnp.dot(a_head, t[:k0, :], preferred_element_type=jnp.float32)
    acc += jnp.dot(a_tail, t[k0:, :], preferred_element_type=jnp.float32)
    return acc


def _fused(nw, k0, ta, tb, tc, na, nb,
           g2_ref, x_ref, p_ref, w_ref, g1_ref, b_ref,
           o_ref, t1_ref, t2_ref):
    i = pl.program_id(0)

    @pl.when(i < na)
    def _phase_a():
        gx = jnp.dot(g2_ref[...], x_ref[...],
                     preferred_element_type=jnp.float32)
        acc = jnp.dot(gx, p_ref[...], preferred_element_type=jnp.float32)
        row = i * ta + jax.lax.broadcasted_iota(jnp.int32, acc.shape, 0)
        t1_ref[pl.ds(i * ta, ta), :] = jnp.where(
            row < nw, acc, 0.0).astype(t1_ref.dtype)

    @pl.when(jnp.logical_and(i >= na, i < na + nb))
    def _phase_b():
        j = i - na
        acc = _masked_k_dot(w_ref, t1_ref[...], nw, k0)
        row = j * tb + jax.lax.broadcasted_iota(jnp.int32, acc.shape, 0)
        t2_ref[pl.ds(j * tb, tb), :] = jnp.where(
            row < nw, acc, 0.0).astype(t2_ref.dtype)

    @pl.when(i >= na + nb)
    def _phase_c():
        acc = _masked_k_dot(g1_ref, t2_ref[...], nw, k0)
        o_ref[...] = acc + b_ref[...]


def kernel(g1, g2, x, weight, p, bias):
    m, nw = g1.shape
    in_dim = x.shape[1]
    out_dim = p.shape[1]

    ta = 256
    tb = 512
    tc = 256
    nwp = _cdiv(nw, 512) * 512
    k0 = (nw // 256) * 256
    na = nwp // ta
    nb = nwp // tb
    nc = m // tc

    def resident(shape):
        return pl.BlockSpec(shape, lambda i: (0, 0))

    out = pl.pallas_call(
        functools.partial(_fused, nw, k0, ta, tb, tc, na, nb),
        out_shape=jax.ShapeDtypeStruct((m, out_dim), jnp.float32),
        grid=(na + nb + nc,),
        in_specs=[
            pl.BlockSpec((ta, m), lambda i: (jnp.minimum(i, na - 1), 0)),
            resident((m, in_dim)),
            resident((in_dim, out_dim)),
            pl.BlockSpec((tb, nwp),
                         lambda i: (jnp.clip(i - na, 0, nb - 1), 0)),
            pl.BlockSpec((tc, nwp),
                         lambda i: (jnp.clip(i - na - nb, 0, nc - 1), 0)),
            resident((1, out_dim)),
        ],
        out_specs=pl.BlockSpec(
            (tc, out_dim), lambda i: (jnp.clip(i - na - nb, 0, nc - 1), 0)),
        scratch_shapes=[
            pltpu.VMEM((nwp, out_dim), jnp.bfloat16),
            pltpu.VMEM((nwp, out_dim), jnp.bfloat16),
        ],
        compiler_params=pltpu.CompilerParams(
            dimension_semantics=("arbitrary",)),
    )(g2, x, p, weight, g1, bias.reshape(1, out_dim))

    return out


# speedup vs baseline: 1.0452x; 1.0452x over previous
import functools

import jax
import jax.numpy as jnp
from jax.experimental import pallas as pl
from jax.experimental.pallas import tpu as pltpu


def _cdiv(a, b):
    return (a + b - 1) // b


def _masked_k_dot(a_ref, t, nw, k0):
    a = a_ref[...].astype(t.dtype)
    a_head = a[:, :k0]
    a_tail = a[:, k0:]
    col = k0 + jax.lax.broadcasted_iota(jnp.int32, a_tail.shape, 1)
    a_tail = jnp.where(col < nw, a_tail, 0)
    acc = jnp.dot(a_head, t[:k0, :], preferred_element_type=jnp.float32)
    acc += jnp.dot(a_tail, t[k0:, :], preferred_element_type=jnp.float32)
    return acc


def _stage_ab(nw, tm, k0, na,
              g2_ref, x_ref, p_ref, w_ref, o_ref, t1_ref):
    i = pl.program_id(0)

    @pl.when(i < na)
    def _phase_a():
        gx = jnp.dot(g2_ref[...], x_ref[...],
                     preferred_element_type=jnp.float32)
        acc = jnp.dot(gx, p_ref[...], preferred_element_type=jnp.float32)
        row = i * tm + jax.lax.broadcasted_iota(jnp.int32, acc.shape, 0)
        t1_ref[pl.ds(i * tm, tm), :] = jnp.where(
            row < nw, acc, 0.0).astype(t1_ref.dtype)

    @pl.when(i >= na)
    def _phase_b():
        j = i - na
        acc = _masked_k_dot(w_ref, t1_ref[...], nw, k0)
        row = j * tm + jax.lax.broadcasted_iota(jnp.int32, acc.shape, 0)
        o_ref[...] = jnp.where(row < nw, acc, 0.0).astype(o_ref.dtype)


def _stage_c(nw, k0, g1_ref, t_ref, b_ref, o_ref):
    o_ref[...] = _masked_k_dot(g1_ref, t_ref[...], nw, k0) + b_ref[...]


def kernel(g1, g2, x, weight, p, bias):
    m, nw = g1.shape
    in_dim = x.shape[1]
    out_dim = p.shape[1]

    tm = 640
    tc = 512
    nwp = _cdiv(nw, 2560) * 2560
    k0 = (nw // 256) * 256
    na = nwp // tm

    def resident(shape):
        return pl.BlockSpec(shape, lambda i: (0, 0))

    t2 = pl.pallas_call(
        functools.partial(_stage_ab, nw, tm, k0, na),
        out_shape=jax.ShapeDtypeStruct((nwp, out_dim), jnp.bfloat16),
        grid=(2 * na,),
        in_specs=[
            pl.BlockSpec((tm, m), lambda i: (jnp.minimum(i, na - 1), 0)),
            resident((m, in_dim)),
            resident((in_dim, out_dim)),
            pl.BlockSpec((tm, nwp),
                         lambda i: (jnp.clip(i - na, 0, na - 1), 0)),
        ],
        out_specs=pl.BlockSpec(
            (tm, out_dim), lambda i: (jnp.clip(i - na, 0, na - 1), 0)),
        scratch_shapes=[pltpu.VMEM((nwp, out_dim), jnp.bfloat16)],
        compiler_params=pltpu.CompilerParams(
            dimension_semantics=("arbitrary",)),
    )(g2, x, p, weight)

    out = pl.pallas_call(
        functools.partial(_stage_c, nw, k0),
        out_shape=jax.ShapeDtypeStruct((m, out_dim), jnp.float32),
        grid=(m // tc,),
        in_specs=[
            pl.BlockSpec((tc, nwp), lambda i: (i, 0)),
            resident((nwp, out_dim)),
            resident((1, out_dim)),
        ],
        out_specs=pl.BlockSpec((tc, out_dim), lambda i: (i, 0)),
        compiler_params=pltpu.CompilerParams(
            dimension_semantics=("parallel",)),
    )(g1, t2, bias.reshape(1, out_dim))

    return out
